# baseline (speedup 1.0000x reference)
import functools

import jax
import jax.numpy as jnp
from jax import lax
from jax.experimental import pallas as pl
from jax.experimental.pallas import tpu as pltpu
from jax.experimental.pallas import tpu_sc as plsc

N = 10000
D = 128
NC = 2
NS = 16
LANES = 16
N_PAD = 10240
RPT = N_PAD // NS
CHUNK = 128
SUP = 8
CPT = 80
NSUP = CPT // SUP
E_PAD = NC * NS * CPT * CHUNK
EC = E_PAD // CHUNK

BK = 1024
GRID = N_PAD // BK

_mesh = plsc.VectorSubcoreMesh(
    core_axis_name="c", subcore_axis_name="s", num_cores=NC, num_subcores=NS
)


@functools.partial(
    pl.kernel,
    out_type=jax.ShapeDtypeStruct((NC, N_PAD), jnp.float32),
    mesh=_mesh,
    scratch_types=[
        pltpu.VMEM((CPT, CHUNK), jnp.int32),
        pltpu.VMEM((CPT, CHUNK), jnp.float32),
        pltpu.VMEM((RPT,), jnp.float32),
        pltpu.VMEM_SHARED((N_PAD,), jnp.float32),
        pltpu.SemaphoreType.DMA,
    ],
)
def _deg_kernel(c_hbm, w_hbm, degp_hbm, cbuf, wbuf, zbuf, degs, ssem):
    cid = lax.axis_index("c")
    sid = lax.axis_index("s")
    wid = cid * NS + sid
    base = wid * CPT

    pltpu.sync_copy(c_hbm.at[pl.ds(base, CPT)], cbuf)
    pltpu.sync_copy(w_hbm.at[pl.ds(base, CPT)], wbuf)

    def _z(i, carry):
        zbuf[pl.ds(i * LANES, LANES)] = jnp.zeros((LANES,), jnp.float32)
        return carry

    lax.fori_loop(0, RPT // LANES, _z, None)
    pltpu.sync_copy(zbuf, degs.at[pl.ds(sid * RPT, RPT)])
    plsc.subcore_barrier()

    GRP = 16

    def _fire(j, carry):
        pltpu.async_copy(wbuf.at[j], degs.at[cbuf.at[j]], ssem, add=True)
        return carry

    def _drain(j, carry):
        pltpu.make_async_copy(wbuf.at[j], degs.at[cbuf.at[j]], ssem).wait()
        return carry

    for g in range(CPT // GRP):
        lax.fori_loop(g * GRP, (g + 1) * GRP, _fire, None)
        lax.fori_loop(g * GRP, (g + 1) * GRP, _drain, None)
    plsc.subcore_barrier()

    @pl.when(sid == 0)
    def _():
        pltpu.sync_copy(degs, degp_hbm.at[cid])


@functools.partial(
    pl.kernel,
    out_type=jax.ShapeDtypeStruct((NC, N_PAD, D), jnp.float32),
    mesh=_mesh,
    scratch_types=[
        pltpu.VMEM((CPT, CHUNK), jnp.int32),
        pltpu.VMEM((CPT, CHUNK), jnp.int32),
        pltpu.VMEM((CPT, CHUNK), jnp.float32),
        pltpu.VMEM((2, CHUNK, D), jnp.float32),
        pltpu.VMEM((2, CHUNK, D), jnp.float32),
        pltpu.VMEM_SHARED((N_PAD, D), jnp.float32),
        pltpu.SemaphoreType.DMA,
        pltpu.SemaphoreType.DMA,
    ],
)
def _scatter_kernel(y_hbm, r_hbm, c_hbm, w_hbm, sp_hbm,
                    rbuf, cbuf, wbuf, gbuf, sbuf, acc, gsem, ssem):
    cid = lax.axis_index("c")
    sid = lax.axis_index("s")
    wid = cid * NS + sid
    base = wid * CPT

    pltpu.sync_copy(r_hbm.at[pl.ds(base, CPT)], rbuf)
    pltpu.sync_copy(c_hbm.at[pl.ds(base, CPT)], cbuf)
    pltpu.sync_copy(w_hbm.at[pl.ds(base, CPT)], wbuf)

    pltpu.async_copy(y_hbm.at[rbuf.at[0]], gbuf.at[0], gsem)
    pltpu.async_copy(y_hbm.at[rbuf.at[1]], gbuf.at[1], gsem)

    def _zr(i, carry):
        for d in range(D // LANES):
            sbuf[0, i, pl.ds(d * LANES, LANES)] = jnp.zeros((LANES,), jnp.float32)
        return carry

    lax.fori_loop(0, CHUNK, _zr, None)
    for k in range(RPT // CHUNK):
        pltpu.sync_copy(sbuf.at[0], acc.at[pl.ds(sid * RPT + k * CHUNK, CHUNK)])
    plsc.subcore_barrier()

    @pl.loop(0, CPT, step=2)
    def _pair(g):
        for b in range(2):
            ch = g + b
            pltpu.make_async_copy(y_hbm.at[rbuf.at[ch]], gbuf.at[b], gsem).wait()

            @pl.when(ch >= 2)
            def _():
                pltpu.make_async_copy(
                    sbuf.at[b], acc.at[cbuf.at[ch - 2]], ssem).wait()

            def _mul(gg, carry):
                wv = wbuf[ch, pl.ds(gg * LANES, LANES)]
                for l in range(LANES):
                    bv = jnp.full((LANES,), wv[l], jnp.float32)
                    e = gg * LANES + l
                    for d in range(D // LANES):
                        sl = pl.ds(d * LANES, LANES)
                        sbuf[b, e, sl] = gbuf[b, e, sl] * bv
                return carry

            lax.fori_loop(0, CHUNK // LANES, _mul, None)
            pltpu.async_copy(sbuf.at[b], acc.at[cbuf.at[ch]], ssem, add=True)

            @pl.when(ch + 2 < CPT)
            def _():
                pltpu.async_copy(y_hbm.at[rbuf.at[ch + 2]], gbuf.at[b], gsem)

    for b in range(2):
        pltpu.make_async_copy(
            sbuf.at[b], acc.at[cbuf.at[CPT - 2 + b]], ssem).wait()
    plsc.subcore_barrier()

    for k in range(RPT // CHUNK):
        r0 = sid * RPT + k * CHUNK
        pltpu.sync_copy(acc.at[pl.ds(r0, CHUNK)], sp_hbm.at[cid, pl.ds(r0, CHUNK)])


def _tc_first(x_ref, w_ref, degp_ref, dis_ref, xw_ref, y_ref):
    deg = 1.0 + degp_ref[0] + degp_ref[1]
    dis = lax.rsqrt(deg)
    xw = jnp.dot(x_ref[...], w_ref[...], preferred_element_type=jnp.float32)
    dis_ref[...] = dis
    xw_ref[...] = xw
    y_ref[...] = xw * dis


def _stage_first(x_pad, W1, degp):
    return pl.pallas_call(
        _tc_first,
        grid=(GRID,),
        in_specs=[
            pl.BlockSpec((BK, D), lambda i: (i, 0)),
            pl.BlockSpec((D, D), lambda i: (0, 0)),
            pl.BlockSpec((NC, BK, 1), lambda i: (0, i, 0)),
        ],
        out_specs=[
            pl.BlockSpec((BK, 1), lambda i: (i, 0)),
            pl.BlockSpec((BK, D), lambda i: (i, 0)),
            pl.BlockSpec((BK, D), lambda i: (i, 0)),
        ],
        out_shape=[
            jax.ShapeDtypeStruct((N_PAD, 1), jnp.float32),
            jax.ShapeDtypeStruct((N_PAD, D), jnp.float32),
            jax.ShapeDtypeStruct((N_PAD, D), jnp.float32),
        ],
    )(x_pad, W1, degp)


def _tc_pre(sp_ref, xw_ref, dis_ref, b_ref, h_ref, ssum_ref, ssq_ref):
    i = pl.program_id(0)
    dis = dis_ref[...]
    h = (sp_ref[0] + sp_ref[1]) * dis + xw_ref[...] * (dis * dis) + b_ref[...]
    ridx = lax.broadcasted_iota(jnp.int32, (BK, 1), 0) + i * BK
    h = h * (ridx < N).astype(jnp.float32)
    h_ref[...] = h

    @pl.when(i == 0)
    def _():
        ssum_ref[...] = jnp.zeros_like(ssum_ref)
        ssq_ref[...] = jnp.zeros_like(ssq_ref)

    ssum_ref[...] += jnp.sum(h, axis=0, keepdims=True)
    ssq_ref[...] += jnp.sum(h * h, axis=0, keepdims=True)


def _stage_pre(sp, xw, dis, b):
    return pl.pallas_call(
        _tc_pre,
        grid=(GRID,),
        in_specs=[
            pl.BlockSpec((NC, BK, D), lambda i: (0, i, 0)),
            pl.BlockSpec((BK, D), lambda i: (i, 0)),
            pl.BlockSpec((BK, 1), lambda i: (i, 0)),
            pl.BlockSpec((1, D), lambda i: (0, 0)),
        ],
        out_specs=[
            pl.BlockSpec((BK, D), lambda i: (i, 0)),
            pl.BlockSpec((1, D), lambda i: (0, 0)),
            pl.BlockSpec((1, D), lambda i: (0, 0)),
        ],
        out_shape=[
            jax.ShapeDtypeStruct((N_PAD, D), jnp.float32),
            jax.ShapeDtypeStruct((1, D), jnp.float32),
            jax.ShapeDtypeStruct((1, D), jnp.float32),
        ],
    )(sp, xw, dis, b)


def _tc_post(h_ref, ssum_ref, ssq_ref, g_ref, be_ref, w_ref, dis_ref,
             xw2_ref, y2_ref):
    mean = ssum_ref[...] * (1.0 / N)
    var = ssq_ref[...] * (1.0 / N) - mean * mean
    inv = lax.rsqrt(var + 1e-5)
    h = (h_ref[...] - mean) * inv * g_ref[...] + be_ref[...]
    h = jnp.maximum(h, 0.0)
    xw2 = jnp.dot(h, w_ref[...], preferred_element_type=jnp.float32)
    xw2_ref[...] = xw2
    y2_ref[...] = xw2 * dis_ref[...]


def _stage_post(h, ssum, ssq, gamma, beta, Wn, dis):
    return pl.pallas_call(
        _tc_post,
        grid=(GRID,),
        in_specs=[
            pl.BlockSpec((BK, D), lambda i: (i, 0)),
            pl.BlockSpec((1, D), lambda i: (0, 0)),
            pl.BlockSpec((1, D), lambda i: (0, 0)),
            pl.BlockSpec((1, D), lambda i: (0, 0)),
            pl.BlockSpec((1, D), lambda i: (0, 0)),
            pl.BlockSpec((D, D), lambda i: (0, 0)),
            pl.BlockSpec((BK, 1), lambda i: (i, 0)),
        ],
        out_specs=[
            pl.BlockSpec((BK, D), lambda i: (i, 0)),
            pl.BlockSpec((BK, D), lambda i: (i, 0)),
        ],
        out_shape=[
            jax.ShapeDtypeStruct((N_PAD, D), jnp.float32),
            jax.ShapeDtypeStruct((N_PAD, D), jnp.float32),
        ],
    )(h, ssum, ssq, gamma, beta, Wn, dis)


def _tc_final(sp_ref, xw_ref, dis_ref, b_ref, out_ref):
    dis = dis_ref[...]
    h = (sp_ref[0] + sp_ref[1]) * dis + xw_ref[...] * (dis * dis) + b_ref[...]
    nrm = jnp.sqrt(jnp.sum(h * h, axis=1, keepdims=True))
    out_ref[...] = h / jnp.maximum(nrm, 1e-12)


def _stage_final(sp, xw, dis, b):
    return pl.pallas_call(
        _tc_final,
        grid=(GRID,),
        in_specs=[
            pl.BlockSpec((NC, BK, D), lambda i: (0, i, 0)),
            pl.BlockSpec((BK, D), lambda i: (i, 0)),
            pl.BlockSpec((BK, 1), lambda i: (i, 0)),
            pl.BlockSpec((1, D), lambda i: (0, 0)),
        ],
        out_specs=pl.BlockSpec((BK, D), lambda i: (i, 0)),
        out_shape=jax.ShapeDtypeStruct((N_PAD, D), jnp.float32),
    )(sp, xw, dis, b)


def kernel(x, edge_index, edge_attr, W1, b1, gamma1, beta1,
           W2, b2, gamma2, beta2, W3, b3):
    r = edge_index[0]
    c = edge_index[1]
    e = r.shape[0]
    pad_e = E_PAD - e
    fill = jnp.arange(pad_e, dtype=jnp.int32)
    r_p = jnp.concatenate([r, fill % N]).reshape(EC, CHUNK)
    c_p = jnp.concatenate([c, fill % N_PAD]).reshape(EC, CHUNK)
    w_p = jnp.concatenate(
        [edge_attr, jnp.zeros((pad_e,), jnp.float32)]).reshape(EC, CHUNK)
    x_pad = jnp.pad(x, ((0, N_PAD - N), (0, 0)))

    degp = _deg_kernel(c_p, w_p).reshape(NC, N_PAD, 1)
    dis, xw1, y1 = _stage_first(x_pad, W1, degp)

    sp1 = _scatter_kernel(y1, r_p, c_p, w_p)
    h1, s1, q1 = _stage_pre(sp1, xw1, dis, b1.reshape(1, D))
    xw2, y2 = _stage_post(h1, s1, q1, gamma1.reshape(1, D),
                          beta1.reshape(1, D), W2, dis)

    sp2 = _scatter_kernel(y2, r_p, c_p, w_p)
    h2, s2, q2 = _stage_pre(sp2, xw2, dis, b2.reshape(1, D))
    xw3, y3 = _stage_post(h2, s2, q2, gamma2.reshape(1, D),
                          beta2.reshape(1, D), W3, dis)

    sp3 = _scatter_kernel(y3, r_p, c_p, w_p)
    out = _stage_final(sp3, xw3, dis, b3.reshape(1, D))
    return out[:N]
